# baseline (speedup 1.0000x reference)
import functools

import jax
import jax.numpy as jnp
from jax import lax
from jax.experimental import pallas as pl
from jax.experimental.pallas import tpu as pltpu
from jax.experimental.pallas import tpu_sc as plsc

EMB = 1024
HID = 1536
NEXP = 8
T = 2048

LANE = 128
TILE = 128
TILE_SHIFT = 7
NT = 4096 // TILE + 7
NT_PAD = 40
NP = NT_PAD * TILE
TE_PAD = 48

NC = 2
NS = 16
NW = NC * NS
TW = T // NW
HTW = TW // 2
NCHUNK = TW // 16

_INV_SQRT2 = 0.7071067811865476


def _router_body(x_ref, wr_ref, br_ref, t1_ref, t2_ref, w1_ref, w2_ref):
    xt = x_ref[...]
    logits = lax.dot_general(xt, wr_ref[...], (((1,), (1,)), ((), ())),
                             preferred_element_type=jnp.float32)
    logits = logits + br_ref[0:1, :]
    lane = lax.broadcasted_iota(jnp.int32, (T, LANE), 1)
    valid = lane < NEXP
    logits = jnp.where(valid, logits, jnp.float32(-1e30))
    m = jnp.max(logits, axis=1, keepdims=True)
    p = jnp.exp(logits - m)
    p = p / jnp.sum(p, axis=1, keepdims=True)
    p = jnp.where(valid, p, jnp.float32(-1.0))
    m1 = jnp.max(p, axis=1, keepdims=True)
    i1 = jnp.min(jnp.where(p >= m1, lane, LANE), axis=1, keepdims=True)
    p2 = jnp.where(lane == i1, jnp.float32(-2.0), p)
    m2 = jnp.max(p2, axis=1, keepdims=True)
    i2 = jnp.min(jnp.where(p2 >= m2, lane, LANE), axis=1, keepdims=True)
    t1_ref[...] = jnp.broadcast_to(i1, (T, LANE))
    t2_ref[...] = jnp.broadcast_to(i2, (T, LANE))
    w1_ref[...] = jnp.broadcast_to(m1, (T, LANE))
    w2_ref[...] = jnp.broadcast_to(m2, (T, LANE))


def _router(x2, Wr, br):
    wr_pad = jnp.zeros((LANE, EMB), jnp.float32).at[:NEXP].set(Wr)
    br_pad = jnp.zeros((8, LANE), jnp.float32).at[:, :NEXP].set(br[None, :])
    shp = jax.ShapeDtypeStruct((T, LANE), jnp.float32)
    shpi = jax.ShapeDtypeStruct((T, LANE), jnp.int32)
    t1f, t2f, w1f, w2f = pl.pallas_call(
        _router_body,
        out_shape=[shpi, shpi, shp, shp],
    )(x2, wr_pad, br_pad)
    return t1f[:, 0], t2f[:, 0], w1f[:, 0], w2f[:, 0]


_SC_MESH = plsc.VectorSubcoreMesh(core_axis_name="c", subcore_axis_name="s")


def _lane_pick(vec, e, iota16):
    return jnp.sum(jnp.where(iota16 == e, vec, 0))


@functools.partial(
    pl.kernel,
    out_type=[
        jax.ShapeDtypeStruct((T,), jnp.int32),
        jax.ShapeDtypeStruct((T,), jnp.int32),
        jax.ShapeDtypeStruct((TE_PAD,), jnp.int32),
        jax.ShapeDtypeStruct((NP, EMB), jnp.float32),
        jax.ShapeDtypeStruct((NP, LANE), jnp.float32),
    ],
    mesh=_SC_MESH,
    scratch_types=[
        pltpu.VMEM((2 * TW,), jnp.int32),
        pltpu.VMEM((2 * TW,), jnp.int32),
        pltpu.VMEM((16,), jnp.int32),
        pltpu.VMEM_SHARED((NW * 16,), jnp.int32),
        pltpu.VMEM((NW * 16,), jnp.int32),
        pltpu.VMEM((TW,), jnp.int32),
        pltpu.VMEM((TW,), jnp.int32),
        pltpu.VMEM((HTW,), jnp.int32),
        pltpu.VMEM((HTW,), jnp.int32),
        pltpu.VMEM((HTW,), jnp.int32),
        pltpu.VMEM((HTW,), jnp.int32),
        pltpu.VMEM((HTW, EMB), jnp.float32),
        pltpu.VMEM((TE_PAD,), jnp.int32),
        pltpu.VMEM((TW + 8,), jnp.float32),
        pltpu.VMEM((TW + 8,), jnp.float32),
        pltpu.VMEM((HTW, LANE), jnp.float32),
        pltpu.VMEM((HTW, LANE), jnp.float32),
        pltpu.VMEM((HTW, LANE), jnp.float32),
        pltpu.VMEM((HTW, LANE), jnp.float32),
        pltpu.SemaphoreType.DMA,
        pltpu.SemaphoreType.DMA,
        pltpu.SemaphoreType.DMA,
        pltpu.SemaphoreType.DMA,
    ],
    compiler_params=pltpu.CompilerParams(needs_layout_passes=False),
)
def _binning(t1_hbm, t2_hbm, w1_hbm, w2_hbm, x_hbm,
             pos1_hbm, pos2_hbm, te_hbm, xs_hbm, pw_hbm,
             t1v, t2v, cvec, cnts_sh, cnts_l, own1, own2,
             p1A, p1B, p2A, p2B, xbuf,
             tev, w1o, w2o, wb1A, wb2A, wb1B, wb2B, sem, sem2, sem3, sem4):
    c = lax.axis_index("c")
    s = lax.axis_index("s")
    w = s * NC + c
    iota16 = lax.iota(jnp.int32, 16)

    for k in range(2):
        g = s + k * NS
        pltpu.sync_copy(t1_hbm.at[pl.ds(g * TW, TW)], t1v.at[pl.ds(k * TW, TW)])
        pltpu.sync_copy(t2_hbm.at[pl.ds(g * TW, TW)], t2v.at[pl.ds(k * TW, TW)])
        cv = jnp.zeros((16,), jnp.int32)
        for ch in range(NCHUNK):
            a = t1v[pl.ds(k * TW + ch * 16, 16)]
            b = t2v[pl.ds(k * TW + ch * 16, 16)]
            for e in range(NEXP):
                n_e = (jnp.sum(jnp.where(a == e, 1, 0))
                       + jnp.sum(jnp.where(b == e, 1, 0)))
                cv = cv + jnp.where(iota16 == e, n_e, 0)
        cvec[...] = cv
        pltpu.sync_copy(cvec, cnts_sh.at[pl.ds(g * 16, 16)])
    plsc.subcore_barrier()
    pltpu.sync_copy(cnts_sh, cnts_l)

    w_vec = iota16 * 0 + w
    zero16 = jnp.zeros((16,), jnp.int32)
    tot = zero16
    pref = zero16
    for g in range(NW):
        row = cnts_l[pl.ds(g * 16, 16)]
        tot = tot + row
        pref = pref + jnp.where(jnp.full((16,), g, jnp.int32) < w_vec,
                                row, zero16)
    tot = jnp.where(iota16 < NEXP, tot, zero16)
    pref = jnp.where(iota16 < NEXP, pref, zero16)
    padded = ((tot + (TILE - 1)) >> TILE_SHIFT) << TILE_SHIFT
    starts = plsc.cumsum(padded) - padded
    base = starts + pref

    pltpu.sync_copy(t1_hbm.at[pl.ds(w * TW, TW)], own1)
    pltpu.sync_copy(t2_hbm.at[pl.ds(w * TW, TW)], own2)
    run = [_lane_pick(base, e, iota16) for e in range(NEXP)]
    p1h = [p1A, p1B]
    p2h = [p2A, p2B]
    for ch in range(NCHUNK):
        a = own1[pl.ds(ch * 16, 16)]
        b = own2[pl.ds(ch * 16, 16)]
        p1c = jnp.zeros((16,), jnp.int32)
        p2c = jnp.zeros((16,), jnp.int32)
        for e in range(NEXP):
            m1 = a == e
            m2 = b == e
            mi = jnp.where(m1 | m2, 1, 0)
            pe = run[e] + plsc.cumsum(mi) - 1
            p1c = jnp.where(m1, pe, p1c)
            p2c = jnp.where(m2, pe, p2c)
            run[e] = run[e] + jnp.sum(mi)
        p1h[ch // 2][pl.ds((ch % 2) * 16, 16)] = p1c
        p2h[ch // 2][pl.ds((ch % 2) * 16, 16)] = p2c
    pltpu.sync_copy(p1A, pos1_hbm.at[pl.ds(w * TW, HTW)])
    pltpu.sync_copy(p1B, pos1_hbm.at[pl.ds(w * TW + HTW, HTW)])
    pltpu.sync_copy(p2A, pos2_hbm.at[pl.ds(w * TW, HTW)])
    pltpu.sync_copy(p2B, pos2_hbm.at[pl.ds(w * TW + HTW, HTW)])

    pltpu.sync_copy(w1_hbm.at[pl.ds(w * TW, TW)], w1o.at[pl.ds(8, TW)])
    pltpu.sync_copy(w2_hbm.at[pl.ds(w * TW, TW)], w2o.at[pl.ds(8, TW)])
    wb1h = [wb1A, wb1B]
    wb2h = [wb2A, wb2B]
    cw = [None, None]
    for h in range(2):
        pltpu.sync_copy(x_hbm.at[pl.ds(w * TW + h * HTW, HTW)], xbuf)
        cx1 = pltpu.async_copy(xbuf, xs_hbm.at[p1h[h]], sem)
        cx2 = pltpu.async_copy(xbuf, xs_hbm.at[p2h[h]], sem2)
        for i in range(HTW):
            idx = jnp.full((16,), h * HTW + i + 8, jnp.int32)
            v1 = plsc.load_gather(w1o, [idx])
            v2 = plsc.load_gather(w2o, [idx])
            for j in range(LANE // 16):
                wb1h[h][i, pl.ds(j * 16, 16)] = v1
                wb2h[h][i, pl.ds(j * 16, 16)] = v2
        cx1.wait()
        cx2.wait()
        cw[h] = (pltpu.async_copy(wb1h[h], pw_hbm.at[p1h[h]], sem3),
                 pltpu.async_copy(wb2h[h], pw_hbm.at[p2h[h]], sem4))
    cw[0][0].wait()
    cw[0][1].wait()
    cw[1][0].wait()
    cw[1][1].wait()

    @pl.when((c == 0) & (s == 0))
    def _te():
        for ch in range(TE_PAD // 16):
            rowbase = (iota16 + ch * 16) * TILE
            te = jnp.full((16,), -1, jnp.int32)
            for e in range(NEXP):
                s_e = _lane_pick(starts, e, iota16)
                p_e = _lane_pick(padded, e, iota16)
                inr = (rowbase >= s_e) & (rowbase < s_e + p_e)
                te = jnp.where(inr, jnp.int32(e), te)
            tev[pl.ds(ch * 16, 16)] = te
        pltpu.sync_copy(tev, te_hbm)


def _ffn_body(te_ref, xs_ref, pw_ref, w1_ref, b1_ref, w2_ref, b2_ref,
              ys_ref, w1bf, w2bf):
    i = pl.program_id(0)
    e = te_ref[i]
    prev = te_ref[jnp.maximum(i - 1, 0)]

    @pl.when((e >= 0) & ((i == 0) | (e != prev)))
    def _cast():
        w1bf[...] = w1_ref[0].astype(jnp.bfloat16)
        w2bf[...] = w2_ref[0].astype(jnp.bfloat16)

    @pl.when(e >= 0)
    def _():
        xt = xs_ref[...].astype(jnp.bfloat16)
        h = lax.dot_general(xt, w1bf[...], (((1,), (1,)), ((), ())),
                            preferred_element_type=jnp.float32)
        h = h + b1_ref[0]
        g = 0.5 * h * (1.0 + lax.erf(h * _INV_SQRT2))
        eo = lax.dot_general(g.astype(jnp.bfloat16), w2bf[...],
                             (((1,), (1,)), ((), ())),
                             preferred_element_type=jnp.float32)
        ys_ref[...] = (eo + b2_ref[0]) * pw_ref[:, 0:1]


def _clamped(te, i):
    e = te[i]
    return jnp.where(e < 0, NEXP - 1, e)


def _ffn(te, xs, pw, W1, b1, W2, b2):
    grid_spec = pltpu.PrefetchScalarGridSpec(
        num_scalar_prefetch=1,
        grid=(NT_PAD,),
        in_specs=[
            pl.BlockSpec((TILE, EMB), lambda i, te: (i, 0)),
            pl.BlockSpec((TILE, LANE), lambda i, te: (i, 0)),
            pl.BlockSpec((1, HID, EMB), lambda i, te: (_clamped(te, i), 0, 0)),
            pl.BlockSpec((1, 1, HID), lambda i, te: (_clamped(te, i), 0, 0)),
            pl.BlockSpec((1, EMB, HID), lambda i, te: (_clamped(te, i), 0, 0)),
            pl.BlockSpec((1, 1, EMB), lambda i, te: (_clamped(te, i), 0, 0)),
        ],
        out_specs=pl.BlockSpec((TILE, EMB), lambda i, te: (i, 0)),
        scratch_shapes=[pltpu.VMEM((HID, EMB), jnp.bfloat16),
                        pltpu.VMEM((EMB, HID), jnp.bfloat16)],
    )
    return pl.pallas_call(
        _ffn_body,
        grid_spec=grid_spec,
        out_shape=jax.ShapeDtypeStruct((NP, EMB), jnp.float32),
        compiler_params=pltpu.CompilerParams(
            dimension_semantics=("arbitrary",)),
    )(te, xs, pw, W1, b1.reshape(NEXP, 1, HID), W2,
      b2.reshape(NEXP, 1, EMB))


CCH = 8


@functools.partial(
    pl.kernel,
    out_type=jax.ShapeDtypeStruct((T, EMB), jnp.float32),
    mesh=_SC_MESH,
    scratch_types=[
        pltpu.VMEM((CCH,), jnp.int32),
        pltpu.VMEM((CCH,), jnp.int32),
        pltpu.VMEM((CCH,), jnp.int32),
        pltpu.VMEM((CCH,), jnp.int32),
        pltpu.VMEM((CCH, EMB), jnp.float32),
        pltpu.VMEM((CCH, EMB), jnp.float32),
        pltpu.VMEM((CCH, EMB), jnp.float32),
        pltpu.VMEM((CCH, EMB), jnp.float32),
        pltpu.VMEM((CCH, EMB), jnp.float32),
        pltpu.SemaphoreType.DMA,
        pltpu.SemaphoreType.DMA,
        pltpu.SemaphoreType.DMA,
        pltpu.SemaphoreType.DMA,
    ],
    compiler_params=pltpu.CompilerParams(needs_layout_passes=False),
)
def _combine(ys_hbm, pos1_hbm, pos2_hbm, out_hbm,
             i1a, i2a, i1b, i2b, r1a, r2a, r1b, r2b, ob,
             s1a, s2a, s1b, s2b):
    c = lax.axis_index("c")
    s = lax.axis_index("s")
    w = s * NC + c
    base = w * TW
    i1 = [i1a, i1b]
    i2 = [i2a, i2b]
    r1 = [r1a, r1b]
    r2 = [r2a, r2b]
    s1 = [s1a, s1b]
    s2 = [s2a, s2b]
    nch = TW // CCH
    pltpu.sync_copy(pos1_hbm.at[pl.ds(base, CCH)], i1[0])
    pltpu.sync_copy(pos2_hbm.at[pl.ds(base, CCH)], i2[0])
    cps = [None, None]
    cps[0] = (pltpu.async_copy(ys_hbm.at[i1[0]], r1[0], s1[0]),
              pltpu.async_copy(ys_hbm.at[i2[0]], r2[0], s2[0]))
    for ch in range(nch):
        cur = ch % 2
        nxt = (ch + 1) % 2
        if ch + 1 < nch:
            off = base + (ch + 1) * CCH
            pltpu.sync_copy(pos1_hbm.at[pl.ds(off, CCH)], i1[nxt])
            pltpu.sync_copy(pos2_hbm.at[pl.ds(off, CCH)], i2[nxt])
            cps[nxt] = (pltpu.async_copy(ys_hbm.at[i1[nxt]], r1[nxt], s1[nxt]),
                        pltpu.async_copy(ys_hbm.at[i2[nxt]], r2[nxt], s2[nxt]))
        cps[cur][0].wait()
        cps[cur][1].wait()
        ra = r1[cur]
        rb = r2[cur]

        def body(j, carry, ra=ra, rb=rb):
            sl = pl.ds(j * 16, 16)
            for t in range(CCH):
                ob[t, sl] = ra[t, sl] + rb[t, sl]
            return carry

        lax.fori_loop(0, EMB // 16, body, 0, unroll=2)
        pltpu.sync_copy(ob, out_hbm.at[pl.ds(base + ch * CCH, CCH)])


def kernel(x, Wr, br, W1, b1, W2, b2):
    B, N, E = x.shape
    x2 = x.reshape(T, E)
    t1, t2, w1, w2 = _router(x2, Wr, br)
    pos1, pos2, te, xs, pw = _binning(t1, t2, w1, w2, x2)
    ys = _ffn(te, xs, pw, W1, b1, W2, b2)
    out = _combine(ys, pos1, pos2)
    return out.reshape(B, N, E)
